# baseline (speedup 1.0000x reference)
import functools

import jax
import jax.numpy as jnp
from jax import lax
from jax.experimental import pallas as pl
from jax.experimental.pallas import tpu as pltpu
from jax.experimental.pallas import tpu_sc as plsc

_B = 16384
_S = 200
_D = 32
_NC = 2
_NS = 16
_NW = _NC * _NS
_V = 1000000
_NVT = 7813
_NST = _S // 8
_NBT = _B // 128
_NT = _NST * _NBT
_TPW = _NT // _NW
_K1N = 246

_CPC = pltpu.CompilerParams(
    use_tc_tiling_on_sc=True,
    needs_layout_passes=False,
    disable_bounds_checks=True,
)
_CPS = pltpu.CompilerParams(
    use_tc_tiling_on_sc=False,
    needs_layout_passes=False,
    disable_bounds_checks=True,
)


def _mesh():
  return plsc.VectorSubcoreMesh(core_axis_name="c", subcore_axis_name="s")


def _k1_untile(cw_t):
  @functools.partial(
      pl.kernel,
      out_type=jax.ShapeDtypeStruct((31250, 8, 128), jnp.float32),
      mesh=_mesh(),
      scratch_types=[
          pltpu.VMEM((2, _D, 128), jnp.float32),
          pltpu.VMEM((2, 4, 8, 128), jnp.float32),
          pltpu.SemaphoreType.DMA,
          pltpu.SemaphoreType.DMA,
          pltpu.SemaphoreType.DMA,
          pltpu.SemaphoreType.DMA,
      ],
      compiler_params=_CPC,
  )
  def k(cw_hbm, out_hbm, src_v, dst_v, s0, s1, w0, w1):
    cid = lax.axis_index("c")
    sid = lax.axis_index("s")
    wid = sid * _NC + cid
    lane = lax.iota(jnp.int32, 16)
    ssem = (s0, s1)
    wsem = (w0, w1)

    def stage(g, b):
      vt = g * _NW + wid

      @pl.when(vt < _NVT)
      def _():
        pltpu.async_copy(
            cw_hbm.at[:, pl.ds(vt * 128, 128)], src_v.at[b], ssem[b])

    def drain_stage(g, b):
      vt = g * _NW + wid

      @pl.when(vt < _NVT)
      def _():
        pltpu.make_async_copy(
            cw_hbm.at[:, pl.ds(vt * 128, 128)], src_v.at[b], ssem[b]).wait()

    def fire_write(g, b):
      vt = g * _NW + wid

      @pl.when(vt < _NVT - 1)
      def _():
        pltpu.async_copy(dst_v.at[b], out_hbm.at[pl.ds(vt * 4, 4)], wsem[b])

      @pl.when(vt == _NVT - 1)
      def _():
        pltpu.async_copy(
            dst_v.at[b, pl.ds(0, 2)],
            out_hbm.at[pl.ds((_NVT - 1) * 4, 2)], wsem[b])

    def drain_write(g, b):
      vt = g * _NW + wid

      @pl.when(vt < _NVT - 1)
      def _():
        pltpu.make_async_copy(
            dst_v.at[b], out_hbm.at[pl.ds(vt * 4, 4)], wsem[b]).wait()

      @pl.when(vt == _NVT - 1)
      def _():
        pltpu.make_async_copy(
            dst_v.at[b, pl.ds(0, 2)],
            out_hbm.at[pl.ds((_NVT - 1) * 4, 2)], wsem[b]).wait()

    def transpose(g, b):
      vt = g * _NW + wid

      @pl.when(vt < _NVT)
      def _():
        bb = jnp.full((16,), b, jnp.int32)
        for q in range(4):
          for r in range(8):
            for h in range(8):
              vl = q * 32 + r * 4 + h // 2
              vals = plsc.load_gather(
                  src_v, [bb, lane + (h % 2) * 16,
                          jnp.full((16,), vl, jnp.int32)])
              dst_v[b, q, r, pl.ds(h * 16, 16)] = vals

    for b in (0, 1):
      stage(b, b)

    def body(i, carry):
      for b in (0, 1):
        g = i * 2 + b
        drain_stage(g, b)

        @pl.when(i > 0)
        def _():
          drain_write(g - 2, b)

        transpose(g, b)
        fire_write(g, b)
        stage(g + 2, b)
      return carry

    lax.fori_loop(0, _K1N // 2, body, 0)
    for b in (0, 1):
      drain_write(_K1N - 2 + b, b)

  return k(cw_t)


def _k2_gather(rm, idx_flat):
  @functools.partial(
      pl.kernel,
      out_type=jax.ShapeDtypeStruct((_NST, 8, 4, _NBT, 8, 128), jnp.float32),
      mesh=_mesh(),
      scratch_types=[
          pltpu.VMEM((2, 1024), jnp.int32),
          pltpu.VMEM((2, 1024, _D), jnp.float32),
          pltpu.VMEM((8, 4, 8, 128), jnp.float32),
          pltpu.SemaphoreType.DMA,
          pltpu.SemaphoreType.DMA,
          pltpu.SemaphoreType.DMA,
          pltpu.SemaphoreType.DMA,
      ],
      compiler_params=_CPS,
  )
  def k(rm_hbm, idx_hbm, out_hbm, idxt_v, rows_v, ot_v, isem, g0, g1, wsem):
    cid = lax.axis_index("c")
    sid = lax.axis_index("s")
    wid = sid * _NC + cid
    lane = lax.iota(jnp.int32, 16)
    bidx = [lane + c0 for c0 in range(0, 128, 16)]
    gsem = (g0, g1)

    def stage_idx(g, b, sem):
      t = g * _NW + wid
      pltpu.async_copy(idx_hbm.at[pl.ds(t * 1024, 1024)], idxt_v.at[b], sem)

    def drain_idx(g, b, sem):
      t = g * _NW + wid
      pltpu.make_async_copy(
          idx_hbm.at[pl.ds(t * 1024, 1024)], idxt_v.at[b], sem).wait()

    def fire_gathers(b):
      for sr in range(8):
        pltpu.async_copy(
            rm_hbm.at[idxt_v.at[b, pl.ds(sr * 128, 128)]],
            rows_v.at[b, pl.ds(sr * 128, 128), :], gsem[b])

    def drain_gathers(b):
      for sr in range(8):
        pltpu.make_async_copy(
            rm_hbm.at[pl.ds(0, 128), :],
            rows_v.at[b, pl.ds(sr * 128, 128), :], gsem[b]).wait()

    def drain_writes():
      for n in range(32):
        pltpu.make_async_copy(
            ot_v.at[0, 0], out_hbm.at[0, 0, 0, 0], wsem).wait()

    stage_idx(0, 0, isem)
    drain_idx(0, 0, isem)
    fire_gathers(0)
    stage_idx(1, 1, isem)
    drain_idx(1, 1, isem)
    fire_gathers(1)

    def body(i, carry):
      for b in (0, 1):
        g = i * 2 + b
        t = g * _NW + wid
        st = t // _NBT
        bt = t % _NBT
        drain_gathers(b)

        @pl.when(g + 2 < _TPW)
        def _():
          stage_idx(g + 2, b, isem)

        @pl.when(g > 0)
        def _():
          drain_writes()

        bb = jnp.full((16,), b, jnp.int32)

        def sb(sr, c):
          srow = jnp.full((16,), sr * 128, jnp.int32)
          for ft in range(4):
            for fr in range(8):
              fvec = jnp.full((16,), ft * 8 + fr, jnp.int32)
              for j in range(8):
                vals = plsc.load_gather(rows_v, [bb, srow + bidx[j], fvec])
                ot_v[sr, ft, fr, pl.ds(j * 16, 16)] = vals
          for ft in range(4):
            pltpu.async_copy(
                ot_v.at[sr, ft], out_hbm.at[st, sr, ft, bt], wsem)
          return c

        lax.fori_loop(0, 8, sb, 0)

        @pl.when(g + 2 < _TPW)
        def _():
          drain_idx(g + 2, b, isem)
          fire_gathers(b)
      return carry

    lax.fori_loop(0, _TPW // 2, body, 0)
    drain_writes()

  return k(rm, idx_flat)


@jax.jit
def _embed(indices, codewords):
  rm = _k1_untile(codewords.T).reshape(_V, _D)
  idx_flat = (
      indices.T.reshape(_NST, 8, _NBT, 128)
      .transpose(0, 2, 1, 3)
      .reshape(-1)
  )
  r6 = _k2_gather(rm, idx_flat)
  out3 = r6.transpose(0, 1, 2, 4, 3, 5).reshape(_S, _D, _B)
  return jnp.transpose(out3, (2, 0, 1))


def kernel(indices, codewords):
  return _embed(indices, codewords)


# speedup vs baseline: 1.3148x; 1.3148x over previous
import functools

import jax
import jax.numpy as jnp
from jax import lax
from jax.experimental import pallas as pl
from jax.experimental.pallas import tpu as pltpu
from jax.experimental.pallas import tpu_sc as plsc

_B = 16384
_S = 200
_D = 32
_TOTAL = _B * _S
_NC = 2
_NS = 16
_NW = _NC * _NS
_RPW = _TOTAL // _NW
_CHUNK = 1024
_NGROUPS = _RPW // _CHUNK
_SUB = 128
_NSUB = _CHUNK // _SUB
_NBUF = 2


@jax.jit
def _sc_gather(idx_flat, table):
  mesh = plsc.VectorSubcoreMesh(core_axis_name="c", subcore_axis_name="s")

  @functools.partial(
      pl.kernel,
      out_type=jax.ShapeDtypeStruct((_TOTAL, _D), jnp.float32),
      mesh=mesh,
      scratch_types=[
          pltpu.VMEM((_NBUF, _CHUNK), jnp.int32),
          pltpu.VMEM((_NBUF, _CHUNK, _D), jnp.float32),
          pltpu.SemaphoreType.DMA,
          pltpu.SemaphoreType.DMA,
          pltpu.SemaphoreType.DMA,
          pltpu.SemaphoreType.DMA,
      ],
      compiler_params=pltpu.CompilerParams(use_tc_tiling_on_sc=False),
  )
  def k(idx_hbm, table_hbm, out_hbm, idx_v, rows_v, g0, g1, w0, w1):
    gsems = (g0, g1)
    wsems = (w0, w1)
    wid = lax.axis_index("s") * _NC + lax.axis_index("c")
    base = wid * _RPW

    def fire_gathers(g, b):
      off = base + g * _CHUNK
      pltpu.sync_copy(idx_hbm.at[pl.ds(off, _CHUNK)], idx_v.at[b])
      for j in range(_NSUB):
        pltpu.async_copy(
            table_hbm.at[idx_v.at[b, pl.ds(j * _SUB, _SUB)]],
            rows_v.at[b, pl.ds(j * _SUB, _SUB)],
            gsems[b],
        )

    def drain_gathers(b):
      for j in range(_NSUB):
        pltpu.make_async_copy(
            table_hbm.at[idx_v.at[b, pl.ds(j * _SUB, _SUB)]],
            rows_v.at[b, pl.ds(j * _SUB, _SUB)],
            gsems[b],
        ).wait()

    def fire_wb(g, b):
      off = base + g * _CHUNK
      pltpu.async_copy(rows_v.at[b], out_hbm.at[pl.ds(off, _CHUNK)], wsems[b])

    def wait_wb(g, b):
      off = base + g * _CHUNK
      pltpu.make_async_copy(
          rows_v.at[b], out_hbm.at[pl.ds(off, _CHUNK)], wsems[b]
      ).wait()

    for b in range(_NBUF):
      fire_gathers(b, b)

    def outer(i, carry):
      g0_ = i * _NBUF
      for b in range(_NBUF):
        g = g0_ + b
        drain_gathers(b)
        fire_wb(g, b)
        wait_wb(g, b)
        fire_gathers(g + _NBUF, b)
      return carry

    lax.fori_loop(0, _NGROUPS // _NBUF - 1, outer, 0)

    gl = _NGROUPS - _NBUF
    for b in range(_NBUF):
      drain_gathers(b)
      fire_wb(gl + b, b)
    for b in range(_NBUF):
      wait_wb(gl + b, b)

  return k(idx_flat, table)


def kernel(indices, codewords):
  idx_flat = indices.reshape(-1).astype(jnp.int32)
  out = _sc_gather(idx_flat, codewords)
  return out.reshape(_B, _S, _D)
